# baseline (speedup 1.0000x reference)
import jax
import jax.numpy as jnp
from jax import lax
from jax.experimental import pallas as pl
from jax.experimental.pallas import tpu as pltpu

N = 8192
D = 128
NCHUNK = 16
CH = N // NCHUNK


def _dma_kernel(pos_ref, e_ref, emb_ref, w_hbm, new_emb_ref, out_hbm,
                sc_ref, sr_ref, stripe_ref, slab_ref, bulk_sems, aux_sems):
    pos = pos_ref[0]
    rb = pl.multiple_of((pos // 8) * 8, 8)
    cb = pl.multiple_of((pos // 128) * 128, 128)

    for k in range(NCHUNK):
        pltpu.make_async_copy(
            w_hbm.at[pl.ds(k * CH, CH), :],
            out_hbm.at[pl.ds(k * CH, CH), :],
            bulk_sems.at[k],
        ).start()

    stripe_in = pltpu.make_async_copy(
        w_hbm.at[:, pl.ds(cb, 128)], stripe_ref, aux_sems.at[0])
    slab_in = pltpu.make_async_copy(
        w_hbm.at[pl.ds(rb, 8), :], slab_ref, aux_sems.at[1])
    stripe_in.start()
    slab_in.start()

    E = emb_ref[...]
    ev = e_ref[...]
    dots_c = lax.dot_general(E, ev, (((1,), (1,)), ((), ())),
                             preferred_element_type=jnp.float32)
    n2_c = jnp.sum(E * E, axis=1, keepdims=True)
    sc_ref[...] = dots_c / (jnp.sqrt(n2_c) + 1e-8)
    dots_r = lax.dot_general(ev, E, (((1,), (1,)), ((), ())),
                             preferred_element_type=jnp.float32)
    ones = jnp.ones((1, D), jnp.float32)
    n2_r = lax.dot_general(ones, E * E, (((1,), (1,)), ((), ())),
                           preferred_element_type=jnp.float32)
    sr_ref[...] = dots_r / (jnp.sqrt(n2_r) + 1e-8)
    rows0 = lax.broadcasted_iota(jnp.int32, (N, D), 0)
    new_emb_ref[...] = jnp.where(rows0 == pos, ev, E)

    stripe_in.wait()
    srow = lax.broadcasted_iota(jnp.int32, (N, 128), 0)
    scol = lax.broadcasted_iota(jnp.int32, (N, 128), 1) + cb
    st = stripe_ref[...]
    st = jnp.where(scol == pos, sc_ref[...], st)
    st = jnp.where(srow == pos, sr_ref[:, pl.ds(cb, 128)], st)
    stripe_ref[...] = st

    slab_in.wait()
    lrow = lax.broadcasted_iota(jnp.int32, (8, N), 0) + rb
    lcol = lax.broadcasted_iota(jnp.int32, (8, N), 1)
    sl = slab_ref[...]
    sl = jnp.where(lcol == pos, sc_ref[pl.ds(rb, 8), :], sl)
    sl = jnp.where(lrow == pos, sr_ref[...], sl)
    slab_ref[...] = sl

    for k in range(NCHUNK):
        pltpu.make_async_copy(
            w_hbm.at[pl.ds(k * CH, CH), :],
            out_hbm.at[pl.ds(k * CH, CH), :],
            bulk_sems.at[k],
        ).wait()

    stripe_out = pltpu.make_async_copy(
        stripe_ref, out_hbm.at[:, pl.ds(cb, 128)], aux_sems.at[0])
    stripe_out.start()
    stripe_out.wait()
    slab_out = pltpu.make_async_copy(
        slab_ref, out_hbm.at[pl.ds(rb, 8), :], aux_sems.at[1])
    slab_out.start()
    slab_out.wait()


def kernel(experience_embeddings, associative_weights, experience,
           temporal_context, position):
    del temporal_context
    pos = jnp.asarray(position, jnp.int32).reshape(1)
    e2 = experience.reshape(1, D)

    new_emb, new_w = pl.pallas_call(
        _dma_kernel,
        out_shape=(jax.ShapeDtypeStruct((N, D), jnp.float32),
                   jax.ShapeDtypeStruct((N, N), jnp.float32)),
        in_specs=[pl.BlockSpec(memory_space=pltpu.SMEM),
                  pl.BlockSpec((1, D), lambda: (0, 0)),
                  pl.BlockSpec((N, D), lambda: (0, 0)),
                  pl.BlockSpec(memory_space=pltpu.MemorySpace.HBM)],
        out_specs=(pl.BlockSpec((N, D), lambda: (0, 0)),
                   pl.BlockSpec(memory_space=pltpu.MemorySpace.HBM)),
        scratch_shapes=[pltpu.VMEM((N, 1), jnp.float32),
                        pltpu.VMEM((1, N), jnp.float32),
                        pltpu.VMEM((N, 128), jnp.float32),
                        pltpu.VMEM((8, N), jnp.float32),
                        pltpu.SemaphoreType.DMA((NCHUNK,)),
                        pltpu.SemaphoreType.DMA((2,))],
    )(pos, e2, experience_embeddings, associative_weights)

    return (new_emb, new_w)


# speedup vs baseline: 46.6088x; 46.6088x over previous
import jax
import jax.numpy as jnp
from jax import lax
from jax.experimental import pallas as pl
from jax.experimental.pallas import tpu as pltpu

N = 8192
D = 128
BLK = 256


def _fused_kernel(pos_ref, e_ref, emb_ref, w_ref, new_emb_ref, out_ref,
                  sc_ref, sr_ref):
    i = pl.program_id(0)
    pos = pos_ref[0]

    @pl.when(i == 0)
    def _():
        E = emb_ref[...]
        ev = e_ref[...]
        dots_c = lax.dot_general(E, ev, (((1,), (1,)), ((), ())),
                                 preferred_element_type=jnp.float32)
        n2_c = jnp.sum(E * E, axis=1, keepdims=True)
        sc_ref[...] = dots_c / (jnp.sqrt(n2_c) + 1e-8)
        dots_r = lax.dot_general(ev, E, (((1,), (1,)), ((), ())),
                                 preferred_element_type=jnp.float32)
        ones = jnp.ones((1, D), jnp.float32)
        n2_r = lax.dot_general(ones, E * E, (((1,), (1,)), ((), ())),
                               preferred_element_type=jnp.float32)
        sr_ref[...] = dots_r / (jnp.sqrt(n2_r) + 1e-8)
        rows0 = lax.broadcasted_iota(jnp.int32, (N, D), 0)
        new_emb_ref[...] = jnp.where(rows0 == pos, ev, E)

    W = w_ref[...]
    rows = lax.broadcasted_iota(jnp.int32, (BLK, N), 0) + i * BLK
    cols = lax.broadcasted_iota(jnp.int32, (BLK, N), 1)
    sc_blk = sc_ref[pl.ds(i * BLK, BLK), :]
    W = jnp.where(cols == pos, sc_blk, W)
    W = jnp.where(rows == pos, sr_ref[...], W)
    out_ref[...] = W


def kernel(experience_embeddings, associative_weights, experience,
           temporal_context, position):
    del temporal_context
    pos = jnp.asarray(position, jnp.int32).reshape(1)
    e2 = experience.reshape(1, D)

    new_emb, new_w = pl.pallas_call(
        _fused_kernel,
        grid=(N // BLK,),
        out_shape=(jax.ShapeDtypeStruct((N, D), jnp.float32),
                   jax.ShapeDtypeStruct((N, N), jnp.float32)),
        in_specs=[pl.BlockSpec(memory_space=pltpu.SMEM),
                  pl.BlockSpec((1, D), lambda i: (0, 0)),
                  pl.BlockSpec((N, D), lambda i: (0, 0)),
                  pl.BlockSpec((BLK, N), lambda i: (i, 0))],
        out_specs=(pl.BlockSpec((N, D), lambda i: (0, 0)),
                   pl.BlockSpec((BLK, N), lambda i: (i, 0))),
        scratch_shapes=[pltpu.VMEM((N, 1), jnp.float32),
                        pltpu.VMEM((1, N), jnp.float32)],
    )(pos, e2, experience_embeddings, associative_weights)

    return (new_emb, new_w)
